# Initial kernel scaffold; baseline (speedup 1.0000x reference)
import jax
import jax.numpy as jnp
from jax.experimental import pallas as pl


def kernel(x, base_W, base_b, router_W, lora_A, lora_B):
    raise NotImplementedError("write your pallas kernel here")


# speedup vs baseline: 3.2418x; 3.2418x over previous
import functools

import jax
import jax.numpy as jnp
from jax.experimental import pallas as pl
from jax.experimental.pallas import tpu as pltpu

T = 8192
D_IN = 4096
D_OUT = 4096
R = 16
E = 8
ER = E * R
_SCALING = 32.0 / 16.0

BT = 512
BD = 512


def _body(x_ref, w_ref, b_ref, rw_ref, aall_ref, ball_ref,
          out_ref, logits_ref, aw_ref):
    j = pl.program_id(1)

    @pl.when(j == 0)
    def _gating():
        x = x_ref[...]
        logits = jax.lax.dot_general(
            x, rw_ref[...], (((1,), (1,)), ((), ())),
            preferred_element_type=jnp.float32)
        logits_ref[...] = logits
        iota_e = jax.lax.broadcasted_iota(jnp.int32, (BT, E), 1)
        v1 = jnp.max(logits, axis=1, keepdims=True)
        i1 = jnp.min(jnp.where(logits == v1, iota_e, E), axis=1, keepdims=True)
        masked = jnp.where(iota_e == i1, -jnp.inf, logits)
        v2 = jnp.max(masked, axis=1, keepdims=True)
        i2 = jnp.min(jnp.where(masked == v2, iota_e, E), axis=1, keepdims=True)
        ed = jnp.exp(v2 - v1)
        denom = 1.0 + ed
        g1 = 1.0 / denom
        g2 = ed / denom
        lane_e = jax.lax.broadcasted_iota(jnp.int32, (BT, ER), 1) // R
        gate_x = (jnp.where(lane_e == i1, g1, 0.0)
                  + jnp.where(lane_e == i2, g2, 0.0)) * _SCALING
        a = jax.lax.dot_general(
            x, aall_ref[...], (((1,), (1,)), ((), ())),
            preferred_element_type=jnp.float32)
        aw_ref[...] = a * gate_x

    acc = jax.lax.dot_general(
        x_ref[...], w_ref[...], (((1,), (1,)), ((), ())),
        preferred_element_type=jnp.float32)
    acc += jnp.dot(aw_ref[...], ball_ref[...],
                   preferred_element_type=jnp.float32)
    out_ref[...] = acc + b_ref[...]


@functools.partial(jax.jit, static_argnames=())
def kernel(x, base_W, base_b, router_W, lora_A, lora_B):
    a_all = lora_A.reshape(ER, D_IN)
    b_all = jnp.transpose(lora_B, (0, 2, 1)).reshape(ER, D_OUT)
    bias = base_b.reshape(1, D_OUT)

    grid = (T // BT, D_OUT // BD)
    out, logits = pl.pallas_call(
        _body,
        grid=grid,
        in_specs=[
            pl.BlockSpec((BT, D_IN), lambda i, j: (i, 0)),
            pl.BlockSpec((BD, D_IN), lambda i, j: (j, 0)),
            pl.BlockSpec((1, BD), lambda i, j: (0, j)),
            pl.BlockSpec((E, D_IN), lambda i, j: (0, 0)),
            pl.BlockSpec((ER, D_IN), lambda i, j: (0, 0)),
            pl.BlockSpec((ER, BD), lambda i, j: (0, j)),
        ],
        out_specs=[
            pl.BlockSpec((BT, BD), lambda i, j: (i, j)),
            pl.BlockSpec((BT, E), lambda i, j: (i, 0)),
        ],
        out_shape=[
            jax.ShapeDtypeStruct((T, D_OUT), jnp.float32),
            jax.ShapeDtypeStruct((T, E), jnp.float32),
        ],
        scratch_shapes=[pltpu.VMEM((BT, ER), jnp.float32)],
        compiler_params=pltpu.CompilerParams(
            dimension_semantics=("parallel", "arbitrary")),
    )(x, base_W, bias, router_W, a_all, b_all)
    return out, logits


# speedup vs baseline: 4.1491x; 1.2799x over previous
import functools

import jax
import jax.numpy as jnp
from jax.experimental import pallas as pl
from jax.experimental.pallas import tpu as pltpu

T = 8192
D_IN = 4096
D_OUT = 4096
R = 16
E = 8
ER = E * R
_SCALING = 32.0 / 16.0

BT = 1024
BD = 512


def _body(x_ref, w_ref, b_ref, rw_ref, aall_ref, ball_ref,
          out_ref, logits_ref, aw_ref):
    j = pl.program_id(1)

    @pl.when(j == 0)
    def _gating():
        x = x_ref[...]
        logits = jax.lax.dot_general(
            x, rw_ref[...], (((1,), (1,)), ((), ())),
            preferred_element_type=jnp.float32)
        logits_ref[...] = logits
        iota_e = jax.lax.broadcasted_iota(jnp.int32, (BT, E), 1)
        v1 = jnp.max(logits, axis=1, keepdims=True)
        i1 = jnp.min(jnp.where(logits == v1, iota_e, E), axis=1, keepdims=True)
        masked = jnp.where(iota_e == i1, -jnp.inf, logits)
        v2 = jnp.max(masked, axis=1, keepdims=True)
        i2 = jnp.min(jnp.where(masked == v2, iota_e, E), axis=1, keepdims=True)
        ed = jnp.exp(v2 - v1)
        denom = 1.0 + ed
        g1 = 1.0 / denom
        g2 = ed / denom
        lane_e = jax.lax.broadcasted_iota(jnp.int32, (BT, ER), 1) // R
        gate_x = (jnp.where(lane_e == i1, g1, 0.0)
                  + jnp.where(lane_e == i2, g2, 0.0)) * _SCALING
        a = jax.lax.dot_general(
            x, aall_ref[...], (((1,), (1,)), ((), ())),
            preferred_element_type=jnp.float32)
        aw_ref[...] = a * gate_x

    acc = jax.lax.dot_general(
        x_ref[...], w_ref[...], (((1,), (1,)), ((), ())),
        preferred_element_type=jnp.float32)
    acc += jnp.dot(aw_ref[...], ball_ref[...],
                   preferred_element_type=jnp.float32)
    out_ref[...] = acc + b_ref[...]


@functools.partial(jax.jit, static_argnames=())
def kernel(x, base_W, base_b, router_W, lora_A, lora_B):
    a_all = lora_A.reshape(ER, D_IN)
    b_all = jnp.transpose(lora_B, (0, 2, 1)).reshape(ER, D_OUT)
    bias = base_b.reshape(1, D_OUT)

    grid = (T // BT, D_OUT // BD)
    out, logits = pl.pallas_call(
        _body,
        grid=grid,
        in_specs=[
            pl.BlockSpec((BT, D_IN), lambda i, j: (i, 0)),
            pl.BlockSpec((BD, D_IN), lambda i, j: (j, 0)),
            pl.BlockSpec((1, BD), lambda i, j: (0, j)),
            pl.BlockSpec((E, D_IN), lambda i, j: (0, 0)),
            pl.BlockSpec((ER, D_IN), lambda i, j: (0, 0)),
            pl.BlockSpec((ER, BD), lambda i, j: (0, j)),
        ],
        out_specs=[
            pl.BlockSpec((BT, BD), lambda i, j: (i, j)),
            pl.BlockSpec((BT, E), lambda i, j: (i, 0)),
        ],
        out_shape=[
            jax.ShapeDtypeStruct((T, D_OUT), jnp.float32),
            jax.ShapeDtypeStruct((T, E), jnp.float32),
        ],
        scratch_shapes=[pltpu.VMEM((BT, ER), jnp.float32)],
        compiler_params=pltpu.CompilerParams(
            dimension_semantics=("parallel", "arbitrary")),
    )(x, base_W, bias, router_W, a_all, b_all)
    return out, logits
